# baseline (speedup 1.0000x reference)
import jax
import jax.numpy as jnp
from jax import lax
from jax.experimental import pallas as pl
from jax.experimental.pallas import tpu as pltpu
from jax.experimental.pallas import tpu_sc as plsc

BATCH = 16384
EMBED = 16
NC = 2
NS = 16
NW = NC * NS
BPW = BATCH // NW
NCHUNK = 4
CHUNK = BPW // NCHUNK


def _ncf_body(uidx_hbm, iidx_hbm, utab_hbm, itab_hbm, wb_hbm, out_hbm,
              uidx_v, iidx_v, urows, irows, srows, out_v, wb_v, sem):
    wid = lax.axis_index("s") * NC + lax.axis_index("c")
    base = wid * BPW

    pltpu.sync_copy(uidx_hbm.at[wid], uidx_v)
    pltpu.sync_copy(iidx_hbm.at[wid], iidx_v)
    pltpu.sync_copy(wb_hbm, wb_v)

    descs = []
    for c in range(NCHUNK):
        dst = urows.at[pl.ds(c * CHUNK, CHUNK)]
        descs.append(pltpu.async_copy(utab_hbm.at[uidx_v.at[c]], dst, sem))
    for c in range(NCHUNK):
        dst = irows.at[pl.ds(c * CHUNK, CHUNK)]
        descs.append(pltpu.async_copy(itab_hbm.at[iidx_v.at[c]], dst, sem))
    for d in descs:
        d.wait()

    wu = wb_v[pl.ds(0, 16)]
    wv = wb_v[pl.ds(16, 16)]
    bsplat = wb_v[pl.ds(32, 16)]

    def p1(g, carry):
        for j in range(16):
            i = g * 16 + j
            srows[pl.ds(i * 16, 16)] = urows[i] * wu + irows[i] * wv
        return carry

    lax.fori_loop(0, BPW // 16, p1, 0)

    iota16 = lax.iota(jnp.int32, 16) * 16

    def p2(g, carry):
        base_idx = g * 256 + iota16
        acc = bsplat
        for e in range(16):
            acc = acc + plsc.load_gather(srows, [base_idx + e])
        out_v[pl.ds(g * 16, 16)] = acc
        return carry

    lax.fori_loop(0, BPW // 16, p2, 0)

    pltpu.sync_copy(out_v, out_hbm.at[pl.ds(base, BPW)])


@jax.jit
def _ncf(uidx, iidx, utab, itab, wb):
    mesh = plsc.VectorSubcoreMesh(core_axis_name="c", subcore_axis_name="s")
    kern = pl.kernel(
        _ncf_body,
        mesh=mesh,
        compiler_params=pltpu.CompilerParams(
            needs_layout_passes=False, use_tc_tiling_on_sc=False),
        out_type=jax.ShapeDtypeStruct((BATCH,), jnp.float32),
        scratch_types=[
            pltpu.VMEM((NCHUNK, CHUNK), jnp.int32),
            pltpu.VMEM((NCHUNK, CHUNK), jnp.int32),
            pltpu.VMEM((BPW, EMBED), jnp.float32),
            pltpu.VMEM((BPW, EMBED), jnp.float32),
            pltpu.VMEM((BPW * EMBED,), jnp.float32),
            pltpu.VMEM((BPW,), jnp.float32),
            pltpu.VMEM((48,), jnp.float32),
            pltpu.SemaphoreType.DMA,
        ],
    )
    return kern(uidx, iidx, utab, itab, wb)


def kernel(user_indices, item_indices, user_table, item_table, W, b):
    uidx = user_indices.astype(jnp.int32).reshape(NW, NCHUNK, CHUNK)
    iidx = item_indices.astype(jnp.int32).reshape(NW, NCHUNK, CHUNK)
    wb = jnp.concatenate([W[:, 0], jnp.full((16,), b[0], jnp.float32)])
    return _ncf(uidx, iidx, user_table, item_table, wb)


# speedup vs baseline: 6.0231x; 6.0231x over previous
import jax
import jax.numpy as jnp
from jax import lax
from jax.experimental import pallas as pl
from jax.experimental.pallas import tpu as pltpu
from jax.experimental.pallas import tpu_sc as plsc

BATCH = 16384
EMBED = 16
NC = 2
NS = 16
NW = NC * NS
BPW = BATCH // NW
C = 16
NCH = BPW // C


def _ncf_body(uidx_hbm, iidx_hbm, utab_t, itab_t, wb_hbm, out_hbm,
              uidx_v, iidx_v, tbuf, out_v, wb_v, sems):
    wid = lax.axis_index("s") * NC + lax.axis_index("c")
    base = wid * BPW

    pltpu.sync_copy(uidx_hbm.at[pl.ds(base, BPW)], uidx_v)
    pltpu.sync_copy(iidx_hbm.at[pl.ds(base, BPW)], iidx_v)
    pltpu.sync_copy(wb_hbm, wb_v)

    wu_vec = wb_v[pl.ds(0, 16)]
    wv_vec = wb_v[pl.ds(16, 16)]
    b_vec = wb_v[pl.ds(32, 16)]
    bias = jnp.zeros((16,), jnp.float32) + b_vec[0]
    iota = lax.iota(jnp.int32, 16)
    row_base = iota * 16

    def fetch(tab, idx_v, ch, buf):
        tvec = lax.shift_right_logical(idx_v[pl.ds(ch * C, C)], 7)
        for j in range(C):
            col0 = pl.multiple_of(tvec[j] * 128, 128)
            pltpu.async_copy(tab.at[:, pl.ds(col0, 128)],
                             tbuf.at[buf, pl.ds(j * 16, 16), :],
                             sems.at[buf])

    def drain(buf):
        for _ in range(C):
            pltpu.make_async_copy(utab_t.at[:, pl.ds(0, 128)],
                                  tbuf.at[buf, pl.ds(0, 16), :],
                                  sems.at[buf]).wait()

    def table_pass(tab, idx_v, w_vec, first):
        fetch(tab, idx_v, 0, 0)

        def chunk_body(ch, carry):
            buf = lax.rem(ch, 2)
            nxt = lax.rem(ch + 1, 2)

            @pl.when(ch + 1 < NCH)
            def _():
                fetch(tab, idx_v, ch + 1, nxt)

            drain(buf)

            lvec = lax.bitwise_and(idx_v[pl.ds(ch * C, C)], 127)
            buf_splat = jnp.zeros((16,), jnp.int32) + buf
            if first:
                acc = bias
            else:
                acc = out_v[pl.ds(ch * C, C)]
            for e in range(EMBED):
                rows = row_base + e
                vals = plsc.load_gather(tbuf, [buf_splat, rows, lvec])
                acc = acc + vals * w_vec[e]
            out_v[pl.ds(ch * C, C)] = acc
            return carry

        lax.fori_loop(0, NCH, chunk_body, 0)

    table_pass(utab_t, uidx_v, wu_vec, True)
    table_pass(itab_t, iidx_v, wv_vec, False)

    pltpu.sync_copy(out_v, out_hbm.at[pl.ds(base, BPW)])


@jax.jit
def _ncf(uidx, iidx, utab_t, itab_t, wb):
    mesh = plsc.VectorSubcoreMesh(core_axis_name="c", subcore_axis_name="s")
    kern = pl.kernel(
        _ncf_body,
        mesh=mesh,
        compiler_params=pltpu.CompilerParams(
            needs_layout_passes=False, use_tc_tiling_on_sc=True),
        out_type=jax.ShapeDtypeStruct((BATCH,), jnp.float32),
        scratch_types=[
            pltpu.VMEM((BPW,), jnp.int32),
            pltpu.VMEM((BPW,), jnp.int32),
            pltpu.VMEM((2, C * 16, 128), jnp.float32),
            pltpu.VMEM((BPW,), jnp.float32),
            pltpu.VMEM((48,), jnp.float32),
            pltpu.SemaphoreType.DMA((2,)),
        ],
    )
    return kern(uidx, iidx, utab_t, itab_t, wb)


def kernel(user_indices, item_indices, user_table, item_table, W, b):
    uidx = user_indices.astype(jnp.int32)
    iidx = item_indices.astype(jnp.int32)
    utab_t = user_table.T
    itab_t = item_table.T
    wb = jnp.concatenate([W[:, 0], b, jnp.zeros((15,), jnp.float32)])
    return _ncf(uidx, iidx, utab_t, itab_t, wb)


# speedup vs baseline: 6.3342x; 1.0517x over previous
import jax
import jax.numpy as jnp
from jax import lax
from jax.experimental import pallas as pl
from jax.experimental.pallas import tpu as pltpu
from jax.experimental.pallas import tpu_sc as plsc

BATCH = 16384
EMBED = 16
NC = 2
NS = 16
NW = NC * NS
BPW = BATCH // NW
C = 16
NCH = BPW // C
NBUF = 3


def _ncf_body(uidx_hbm, iidx_hbm, utab_t, itab_t, wb_hbm, out_hbm,
              uidx_v, iidx_v, tbuf, out_v, wb_v, sems):
    wid = lax.axis_index("s") * NC + lax.axis_index("c")
    base = wid * BPW

    pltpu.sync_copy(uidx_hbm.at[pl.ds(base, BPW)], uidx_v)
    pltpu.sync_copy(iidx_hbm.at[pl.ds(base, BPW)], iidx_v)
    pltpu.sync_copy(wb_hbm, wb_v)

    wu_vec = wb_v[pl.ds(0, 16)]
    wv_vec = wb_v[pl.ds(16, 16)]
    b_vec = wb_v[pl.ds(32, 16)]
    bias = jnp.zeros((16,), jnp.float32) + b_vec[0]
    iota = lax.iota(jnp.int32, 16)
    row_base = iota * 16

    def fetch(tab, idx_v, ch, buf):
        tvec = lax.shift_right_logical(idx_v[pl.ds(ch * C, C)], 7)
        for j in range(C):
            col0 = pl.multiple_of(tvec[j] * 128, 128)
            pltpu.async_copy(tab.at[:, pl.ds(col0, 128)],
                             tbuf.at[buf, pl.ds(j * 16, 16), :],
                             sems.at[buf])

    def drain(buf):
        for _ in range(C):
            pltpu.make_async_copy(utab_t.at[:, pl.ds(0, 128)],
                                  tbuf.at[buf, pl.ds(0, 16), :],
                                  sems.at[buf]).wait()

    def table_pass(tab, idx_v, w_vec, first):
        fetch(tab, idx_v, 0, 0)
        fetch(tab, idx_v, 1, 1)

        def chunk_body(ch, carry):
            buf = lax.rem(ch, NBUF)
            nxt = lax.rem(ch + 2, NBUF)

            @pl.when(ch + 2 < NCH)
            def _():
                fetch(tab, idx_v, ch + 2, nxt)

            drain(buf)

            lvec = lax.bitwise_and(idx_v[pl.ds(ch * C, C)], 127)
            buf_splat = jnp.zeros((16,), jnp.int32) + buf
            if first:
                acc = bias
            else:
                acc = out_v[pl.ds(ch * C, C)]
            for e in range(EMBED):
                rows = row_base + e
                vals = plsc.load_gather(tbuf, [buf_splat, rows, lvec])
                acc = acc + vals * w_vec[e]
            out_v[pl.ds(ch * C, C)] = acc
            return carry

        lax.fori_loop(0, NCH, chunk_body, 0)

    table_pass(utab_t, uidx_v, wu_vec, True)
    table_pass(itab_t, iidx_v, wv_vec, False)

    pltpu.sync_copy(out_v, out_hbm.at[pl.ds(base, BPW)])


@jax.jit
def _ncf(uidx, iidx, utab_t, itab_t, wb):
    mesh = plsc.VectorSubcoreMesh(core_axis_name="c", subcore_axis_name="s")
    kern = pl.kernel(
        _ncf_body,
        mesh=mesh,
        compiler_params=pltpu.CompilerParams(
            needs_layout_passes=False, use_tc_tiling_on_sc=True),
        out_type=jax.ShapeDtypeStruct((BATCH,), jnp.float32),
        scratch_types=[
            pltpu.VMEM((BPW,), jnp.int32),
            pltpu.VMEM((BPW,), jnp.int32),
            pltpu.VMEM((NBUF, C * 16, 128), jnp.float32),
            pltpu.VMEM((BPW,), jnp.float32),
            pltpu.VMEM((48,), jnp.float32),
            pltpu.SemaphoreType.DMA((NBUF,)),
        ],
    )
    return kern(uidx, iidx, utab_t, itab_t, wb)


def kernel(user_indices, item_indices, user_table, item_table, W, b):
    uidx = user_indices.astype(jnp.int32)
    iidx = item_indices.astype(jnp.int32)
    utab_t = user_table.T
    itab_t = item_table.T
    wb = jnp.concatenate([W[:, 0], b, jnp.zeros((15,), jnp.float32)])
    return _ncf(uidx, iidx, utab_t, itab_t, wb)
